# baseline (speedup 1.0000x reference)
import functools

import jax
import jax.numpy as jnp
import numpy as np
from jax import lax
from jax.experimental import pallas as pl
from jax.experimental.pallas import tpu as pltpu
from jax.experimental.pallas import tpu_sc as plsc

_F32 = jnp.float32
_I32 = jnp.int32

_B, _NA, _NO = 64, 3, 12
_NT = 2000
_NCAND = 5 * _NA * _NT
_NW = 32
_CHUNKS = 64
_CPT = _CHUNKS * 16
_M = _CPT * _NW
_NBATCH = 8
_BCH = _CHUNKS // _NBATCH
_BROWS = _BCH * 16 * 2
_GAIN = 20.0
_CELLS = _B * _NA * 20 * 20
_HW = [(80, 80), (40, 40), (20, 20)]
_NCELLS = [_B * _NA * h * w for h, w in _HW]
_ANCH = [[(10.0, 13.0), (16.0, 30.0), (33.0, 23.0)],
         [(30.0, 61.0), (62.0, 45.0), (59.0, 119.0)],
         [(116.0, 90.0), (156.0, 198.0), (373.0, 326.0)]]
_BALANCE = [4.0, 1.0, 0.4]
_BOX_W, _OBJ_W, _CLS_W = 0.05, 1.0, 0.5


def _splat_i32(val):
    return jnp.full((16,), val, dtype=_I32)


def _phase_a_body(t_hbm, pf0, pf1, pf2,
                  cell_o, x4v_o, cls_o, part_o,
                  tvm, idxw, aoff, psb,
                  atbx, atby, atw, ath, aaw, aah, aval, acell,
                  x4vb, clsb, stage, dsem):
    wid = lax.axis_index("s") * 2 + lax.axis_index("c")
    base = wid * _CPT
    lane = lax.iota(_I32, 16)
    zf = jnp.zeros((16,), _F32)

    pltpu.sync_copy(t_hbm, tvm)

    for s, pf in enumerate((pf0, pf1, pf2)):
        H, W = _HW[s]
        wmax = (_B * _NA * H * W * _NO) // 128 - 1
        (a0w, a0h), (a1w, a1h), (a2w, a2h) = _ANCH[s]

        @pl.loop(0, _CHUNKS)
        def _pass1(j):
            sl = pl.ds(j * 16, 16)
            q = base + j * 16 + lane
            o = lax.div(q, 6000)
            rem = q - o * 6000
            a = lax.div(rem, 2000)
            t = rem - a * 2000
            t6 = t * 6
            tb = plsc.load_gather(tvm, [t6])
            tc = plsc.load_gather(tvm, [t6 + 1])
            tx = plsc.load_gather(tvm, [t6 + 2]) * _GAIN
            ty = plsc.load_gather(tvm, [t6 + 3]) * _GAIN
            tw = plsc.load_gather(tvm, [t6 + 4]) * _GAIN
            th = plsc.load_gather(tvm, [t6 + 5]) * _GAIN
            aw = jnp.where(a == 0, a0w, jnp.where(a == 1, a1w, a2w)).astype(_F32)
            ah = jnp.where(a == 0, a0h, jnp.where(a == 1, a1h, a2h)).astype(_F32)
            rw = tw / aw
            rh = th / ah
            mx = jnp.maximum(jnp.maximum(rw, 1.0 / rw), jnp.maximum(rh, 1.0 / rh))
            jmv = mx < 4.0
            jj = (lax.rem(tx, jnp.float32(1.0)) < 0.5) & (tx > 1.0)
            kk = (lax.rem(ty, jnp.float32(1.0)) < 0.5) & (ty > 1.0)
            gxx = W - tx
            gyy = H - ty
            ll = (lax.rem(gxx, jnp.float32(1.0)) < 0.5) & (gxx > 1.0)
            mm = (lax.rem(gyy, jnp.float32(1.0)) < 0.5) & (gyy > 1.0)
            jmask = ((o == 0) | ((o == 1) & jj) | ((o == 2) & kk)
                     | ((o == 3) & ll) | ((o == 4) & mm))
            valid = jmask & jmv
            offx = jnp.where(o == 1, 0.5, jnp.where(o == 3, -0.5, 0.0)).astype(_F32)
            offy = jnp.where(o == 2, 0.5, jnp.where(o == 4, -0.5, 0.0)).astype(_F32)
            gijx = (tx - offx).astype(_I32)
            gijy = (ty - offy).astype(_I32)
            gi = jnp.clip(gijx, 0, W - 1)
            gj = jnp.clip(gijy, 0, H - 1)
            bi = tb.astype(_I32)
            ba = bi * 3 + a
            rfull = (ba * H + gj) * W + gi
            e0 = rfull * _NO
            w0 = lax.shift_right_logical(e0, 7)
            w1 = jnp.minimum(w0 + 1, wmax)
            cell = (ba * 20 + jnp.minimum(gj, 19)) * 20 + jnp.minimum(gi, 19)
            plsc.store_scatter(idxw, [j * 32 + 2 * lane], w0)
            plsc.store_scatter(idxw, [j * 32 + 2 * lane + 1], w1)
            aoff[sl] = e0 - w0 * 128
            atbx[sl] = tx - gijx.astype(_F32)
            atby[sl] = ty - gijy.astype(_F32)
            atw[sl] = tw
            ath[sl] = th
            aaw[sl] = aw
            aah[sl] = ah
            aval[sl] = jnp.where(valid, 1.0, 0.0).astype(_F32)
            acell[sl] = jnp.where(valid, cell, -1)
            clsb[7, sl] = jnp.where(valid, tc, -1.0).astype(_F32)

        def _fire(b):
            slot = b % 2
            descs = []
            for (lo, n) in ((0, 128), (128, 128)):
                isl = pl.ds(b * _BROWS + lo, n)
                dsl = pl.ds(slot * _BROWS + lo, n)
                descs.append(
                    pltpu.async_copy(pf.at[idxw.at[isl]], psb.at[dsl, :], dsem))
            return descs

        pending = _fire(0)

        def _pass2(args):
            j, carry = args
            lbox_acc, cnt_acc = carry
            sl = pl.ds(j * 16, 16)
            slot_base = lax.rem(lax.div(j, _BCH), 2) * _BROWS
            lc = (j - lax.div(j, _BCH) * _BCH) * 16 + lane
            off = aoff[sl]
            rbase = slot_base + lc * 2
            p = []
            for k in range(_NO):
                fk = off + k
                row = rbase + lax.shift_right_logical(fk, 7)
                col = fk & 127
                p.append(plsc.load_gather(psb, [row, col]))
            sg = lambda z: 1.0 / (1.0 + jnp.exp(-z))
            aw = aaw[sl]
            ah = aah[sl]
            px = sg(p[0]) * 2.0 - 0.5
            py = sg(p[1]) * 2.0 - 0.5
            sw = sg(p[2]) * 2.0
            sh = sg(p[3]) * 2.0
            pw = sw * sw * aw
            ph = sh * sh * ah
            tbx = atbx[sl]
            tby = atby[sl]
            tw = atw[sl]
            th = ath[sl]
            eps = jnp.float32(1e-7)
            b1x1, b1x2 = px - pw * 0.5, px + pw * 0.5
            b1y1, b1y2 = py - ph * 0.5, py + ph * 0.5
            b2x1, b2x2 = tbx - tw * 0.5, tbx + tw * 0.5
            b2y1, b2y2 = tby - th * 0.5, tby + th * 0.5
            iw = jnp.maximum(jnp.minimum(b1x2, b2x2) - jnp.maximum(b1x1, b2x1), 0.0)
            ih = jnp.maximum(jnp.minimum(b1y2, b2y2) - jnp.maximum(b1y1, b2y1), 0.0)
            inter = iw * ih
            w1_, h1_ = b1x2 - b1x1, b1y2 - b1y1 + eps
            w2_, h2_ = b2x2 - b2x1, b2y2 - b2y1 + eps
            union = w1_ * h1_ + w2_ * h2_ - inter + eps
            iou = inter / union
            vf = aval[sl]
            lbox_acc = lbox_acc + (1.0 - iou) * vf
            cnt_acc = cnt_acc + vf
            v = jnp.maximum(iou, 0.0)
            x4vb[sl] = p[4] * v * vf
            for k in range(7):
                clsb[k, sl] = p[5 + k]
            return lbox_acc, cnt_acc

        carry = (zf, zf)
        for b in range(_NBATCH):
            if b + 1 < _NBATCH:
                nxt = _fire(b + 1)
            else:
                nxt = None
            for cp in pending:
                cp.wait()
            pending = nxt

            @pl.loop(0, _BCH, init_carry=carry)
            def _batch(i, c):
                return _pass2((b * _BCH + i, c))

            carry = _batch
        lbox_acc, cnt_acc = carry

        pltpu.sync_copy(acell, cell_o.at[pl.ds(s * _M + base, _CPT)])
        pltpu.sync_copy(x4vb, x4v_o.at[pl.ds(s * _M + base, _CPT)])
        for k in range(8):
            pltpu.sync_copy(clsb.at[k],
                            cls_o.at[pl.ds((s * 8 + k) * _M + base, _CPT)])
        stage[pl.ds(0, 16)] = lbox_acc
        stage[pl.ds(16, 16)] = cnt_acc
        for z in range(2, 8):
            stage[pl.ds(z * 16, 16)] = zf
        pltpu.sync_copy(stage, part_o.at[pl.ds((s * _NW + wid) * 128, 128)])


def _phase_b_body(cell_hbm, x4v_hbm, corr_o, L, cbuf, vbuf, stg):
    wid = lax.axis_index("s") * 2 + lax.axis_index("c")
    for z in range(8):
        stg[pl.ds(z * 16, 16)] = jnp.zeros((16,), _F32)
    pltpu.sync_copy(stg, corr_o.at[pl.ds(wid * 128, 128)])

    @pl.when(wid < 3)
    def _work():
        @pl.loop(0, _CELLS // 16)
        def _zero(i):
            L[pl.ds(i * 16, 16)] = jnp.zeros((16,), _F32)

        nch = 8
        csz = _M // nch
        for ch in range(nch):
            pltpu.sync_copy(cell_hbm.at[pl.ds(wid * _M + ch * csz, csz)], cbuf)
            pltpu.sync_copy(x4v_hbm.at[pl.ds(wid * _M + ch * csz, csz)], vbuf)

            @pl.loop(0, csz // 16)
            def _scatter(i):
                sl = pl.ds(i * 16, 16)
                c = cbuf[sl]
                vv = vbuf[sl]
                plsc.store_scatter(L, [c], vv, mask=c >= 0)

        @pl.loop(0, _CELLS // 16, init_carry=jnp.zeros((16,), _F32))
        def _reduce(i, acc):
            return acc + L[pl.ds(i * 16, 16)]

        stg[pl.ds(0, 16)] = _reduce
        pltpu.sync_copy(stg, corr_o.at[pl.ds(wid * 128, 128)])


def _softsum_channel4(pflat2d, rows_blk):
    rows, width = pflat2d.shape
    steps = rows // rows_blk

    def body(x_ref, o_ref):
        @pl.when(pl.program_id(0) == 0)
        def _init():
            o_ref[...] = jnp.zeros((1, 1), _F32)

        x = x_ref[...]
        col = lax.broadcasted_iota(_I32, x.shape, 1)
        sp = jnp.maximum(x, 0.0) + jnp.log1p(jnp.exp(-jnp.abs(x)))
        o_ref[...] += jnp.sum(jnp.where(col % _NO == 4, sp, 0.0)).reshape(1, 1)

    out = pl.pallas_call(
        body,
        grid=(steps,),
        in_specs=[pl.BlockSpec((rows_blk, width), lambda i: (i, 0))],
        out_specs=pl.BlockSpec((1, 1), lambda i: (0, 0)),
        out_shape=jax.ShapeDtypeStruct((1, 1), _F32),
    )(pflat2d)
    return out[0, 0]


def _cls_sums(cls_rows):
    def body(x_ref, o_ref):
        x = x_ref[0]
        l = x[:7, :]
        extra = x[7, :]
        vf = jnp.where(extra >= 0.0, 1.0, 0.0).astype(_F32)
        c = jnp.maximum(extra, 0.0).astype(_I32)
        oh = (lax.broadcasted_iota(_I32, (7, _M), 0) == c[None, :]).astype(_F32)
        ce = jnp.maximum(l, 0.0) - l * oh + jnp.log1p(jnp.exp(-jnp.abs(l)))
        ssum = jnp.sum(ce * vf[None, :])
        i = pl.program_id(0)

        @pl.when(i == 0)
        def _init():
            o_ref[...] = jnp.zeros((8, 128), _F32)

        rowi = lax.broadcasted_iota(_I32, (8, 128), 0)
        lanei = lax.broadcasted_iota(_I32, (8, 128), 1)
        o_ref[...] += jnp.where((rowi == i) & (lanei == 0), ssum, 0.0)

    out = pl.pallas_call(
        body,
        grid=(3,),
        in_specs=[pl.BlockSpec((1, 8, _M), lambda i: (i, 0, 0))],
        out_specs=pl.BlockSpec((8, 128), lambda i: (0, 0)),
        out_shape=jax.ShapeDtypeStruct((8, 128), _F32),
    )(cls_rows)
    return out[:3, 0]


def kernel(p3, p4, p5, targets, img_size):
    preds = (p3, p4, p5)
    pviews = tuple(p.reshape(-1, 128) for p in preds)
    tflat = targets.reshape(-1).astype(_F32)
    tflat = jnp.concatenate([tflat, jnp.zeros((12032 - 6 * _NT,), _F32)])

    mesh = plsc.VectorSubcoreMesh(core_axis_name="c", subcore_axis_name="s")
    phase_a = pl.kernel(
        _phase_a_body,
        out_type=[
            jax.ShapeDtypeStruct((3 * _M,), _I32),
            jax.ShapeDtypeStruct((3 * _M,), _F32),
            jax.ShapeDtypeStruct((3 * 8 * _M,), _F32),
            jax.ShapeDtypeStruct((3 * _NW * 128,), _F32),
        ],
        mesh=mesh,
        compiler_params=pltpu.CompilerParams(needs_layout_passes=False),
        scratch_types=[
            pltpu.VMEM((12032,), _F32),
            pltpu.VMEM((_CHUNKS * 32,), _I32),
            pltpu.VMEM((_CPT,), _I32),
            pltpu.VMEM((2 * _BROWS, 128), _F32),
            pltpu.VMEM((_CPT,), _F32),
            pltpu.VMEM((_CPT,), _F32),
            pltpu.VMEM((_CPT,), _F32),
            pltpu.VMEM((_CPT,), _F32),
            pltpu.VMEM((_CPT,), _F32),
            pltpu.VMEM((_CPT,), _F32),
            pltpu.VMEM((_CPT,), _F32),
            pltpu.VMEM((_CPT,), _I32),
            pltpu.VMEM((_CPT,), _F32),
            pltpu.VMEM((8, _CPT), _F32),
            pltpu.VMEM((128,), _F32),
            pltpu.SemaphoreType.DMA,
        ],
    )
    cells, x4v, cls_flat, part_flat = phase_a(tflat, *pviews)
    cls_rows = cls_flat.reshape(3, 8, _M)
    part = part_flat.reshape(3, _NW, 128)

    phase_b = pl.kernel(
        _phase_b_body,
        out_type=[jax.ShapeDtypeStruct((_NW * 128,), _F32)],
        mesh=mesh,
        compiler_params=pltpu.CompilerParams(needs_layout_passes=False),
        scratch_types=[
            pltpu.VMEM((_CELLS,), _F32),
            pltpu.VMEM((_M // 8,), _I32),
            pltpu.VMEM((_M // 8,), _F32),
            pltpu.VMEM((128,), _F32),
        ],
    )
    (corr,) = phase_b(cells, x4v)
    corr = corr.reshape(_NW, 128)

    softs = [
        _softsum_channel4(p3.reshape(3840, 3840), 240),
        _softsum_channel4(p4.reshape(960, 3840), 240),
        _softsum_channel4(p5.reshape(240, 3840), 240),
    ]
    clsums = _cls_sums(cls_rows)

    lbox_s = part[:, :, 0:16].sum(axis=(1, 2))
    cnt_s = part[:, :, 16:32].sum(axis=(1, 2))
    corr_s = corr[:3, :16].sum(axis=1)
    denom = jnp.maximum(cnt_s, 1.0)
    pos = cnt_s > 0
    lbox = jnp.where(pos, lbox_s / denom, 0.0).sum() * _BOX_W
    lcls = jnp.where(pos, clsums / (denom * 7.0), 0.0).sum() * _CLS_W
    lobj = sum(
        (softs[s] - corr_s[s]) / _NCELLS[s] * _BALANCE[s] for s in range(3)
    ) * _OBJ_W
    total = lbox + lobj + lcls
    total = total + jnp.asarray(img_size, dtype=total.dtype) * 0.0
    return (total, jnp.asarray(lbox, _F32), jnp.asarray(lobj, _F32),
            jnp.asarray(lcls, _F32))


# speedup vs baseline: 1.5596x; 1.5596x over previous
import functools

import jax
import jax.numpy as jnp
import numpy as np
from jax import lax
from jax.experimental import pallas as pl
from jax.experimental.pallas import tpu as pltpu
from jax.experimental.pallas import tpu_sc as plsc

_F32 = jnp.float32
_I32 = jnp.int32

_B, _NA, _NO = 64, 3, 12
_NT = 2000
_NCAND = 5 * _NA * _NT
_NW = 32
_CHUNKS = 64
_CPT = _CHUNKS * 16
_M = _CPT * _NW
_NBATCH = 8
_BCH = _CHUNKS // _NBATCH
_BROWS = _BCH * 16 * 2
_GAIN = 20.0
_CELLS = _B * _NA * 20 * 20
_HW = [(80, 80), (40, 40), (20, 20)]
_NCELLS = [_B * _NA * h * w for h, w in _HW]
_ANCH = [[(10.0, 13.0), (16.0, 30.0), (33.0, 23.0)],
         [(30.0, 61.0), (62.0, 45.0), (59.0, 119.0)],
         [(116.0, 90.0), (156.0, 198.0), (373.0, 326.0)]]
_BALANCE = [4.0, 1.0, 0.4]
_BOX_W, _OBJ_W, _CLS_W = 0.05, 1.0, 0.5


def _splat_i32(val):
    return jnp.full((16,), val, dtype=_I32)


def _phase_a_body(t_hbm, pf0, pf1, pf2,
                  cell_o, x4v_o, cls_o, part_o,
                  tvm, idxw, aoff, psb,
                  atbx, atby, atw, ath, aaw, aah, aval, acell,
                  x4vb, clsb, stage, dsem):
    wid = lax.axis_index("s") * 2 + lax.axis_index("c")
    base = wid * _CPT
    lane = lax.iota(_I32, 16)
    zf = jnp.zeros((16,), _F32)

    pltpu.sync_copy(t_hbm, tvm)

    for s, pf in enumerate((pf0, pf1, pf2)):
        H, W = _HW[s]
        wmax = (_B * _NA * H * W * _NO) // 128 - 1
        (a0w, a0h), (a1w, a1h), (a2w, a2h) = _ANCH[s]

        @pl.loop(0, _CHUNKS)
        def _pass1(j):
            sl = pl.ds(j * 16, 16)
            q = base + j * 16 + lane
            o = lax.div(q, 6000)
            rem = q - o * 6000
            a = lax.div(rem, 2000)
            t = rem - a * 2000
            t6 = t * 6
            tb = plsc.load_gather(tvm, [t6])
            tc = plsc.load_gather(tvm, [t6 + 1])
            tx = plsc.load_gather(tvm, [t6 + 2]) * _GAIN
            ty = plsc.load_gather(tvm, [t6 + 3]) * _GAIN
            tw = plsc.load_gather(tvm, [t6 + 4]) * _GAIN
            th = plsc.load_gather(tvm, [t6 + 5]) * _GAIN
            aw = jnp.where(a == 0, a0w, jnp.where(a == 1, a1w, a2w)).astype(_F32)
            ah = jnp.where(a == 0, a0h, jnp.where(a == 1, a1h, a2h)).astype(_F32)
            rw = tw / aw
            rh = th / ah
            mx = jnp.maximum(jnp.maximum(rw, 1.0 / rw), jnp.maximum(rh, 1.0 / rh))
            jmv = mx < 4.0
            jj = (lax.rem(tx, jnp.float32(1.0)) < 0.5) & (tx > 1.0)
            kk = (lax.rem(ty, jnp.float32(1.0)) < 0.5) & (ty > 1.0)
            gxx = W - tx
            gyy = H - ty
            ll = (lax.rem(gxx, jnp.float32(1.0)) < 0.5) & (gxx > 1.0)
            mm = (lax.rem(gyy, jnp.float32(1.0)) < 0.5) & (gyy > 1.0)
            jmask = ((o == 0) | ((o == 1) & jj) | ((o == 2) & kk)
                     | ((o == 3) & ll) | ((o == 4) & mm))
            valid = jmask & jmv
            offx = jnp.where(o == 1, 0.5, jnp.where(o == 3, -0.5, 0.0)).astype(_F32)
            offy = jnp.where(o == 2, 0.5, jnp.where(o == 4, -0.5, 0.0)).astype(_F32)
            gijx = (tx - offx).astype(_I32)
            gijy = (ty - offy).astype(_I32)
            gi = jnp.clip(gijx, 0, W - 1)
            gj = jnp.clip(gijy, 0, H - 1)
            bi = tb.astype(_I32)
            ba = bi * 3 + a
            rfull = (ba * H + gj) * W + gi
            e0 = rfull * _NO
            w0 = lax.shift_right_logical(e0, 7)
            w1 = jnp.minimum(w0 + 1, wmax)
            cell = (ba * 20 + jnp.minimum(gj, 19)) * 20 + jnp.minimum(gi, 19)
            plsc.store_scatter(idxw, [j * 32 + 2 * lane], w0)
            plsc.store_scatter(idxw, [j * 32 + 2 * lane + 1], w1)
            aoff[sl] = e0 - w0 * 128
            atbx[sl] = tx - gijx.astype(_F32)
            atby[sl] = ty - gijy.astype(_F32)
            atw[sl] = tw
            ath[sl] = th
            aaw[sl] = aw
            aah[sl] = ah
            aval[sl] = jnp.where(valid, 1.0, 0.0).astype(_F32)
            acell[sl] = jnp.where(valid, cell, -1)
            clsb[7, sl] = jnp.where(valid, tc, -1.0).astype(_F32)

        def _fire(b):
            slot = b % 2
            descs = []
            for (lo, n) in ((0, 128), (128, 128)):
                isl = pl.ds(b * _BROWS + lo, n)
                dsl = pl.ds(slot * _BROWS + lo, n)
                descs.append(
                    pltpu.async_copy(pf.at[idxw.at[isl]], psb.at[dsl, :], dsem))
            return descs

        pending = _fire(0)

        def _pass2(args):
            j, carry = args
            lbox_acc, cnt_acc = carry
            sl = pl.ds(j * 16, 16)
            slot_base = lax.rem(lax.div(j, _BCH), 2) * _BROWS
            lc = (j - lax.div(j, _BCH) * _BCH) * 16 + lane
            off = aoff[sl]
            rbase = slot_base + lc * 2
            p = []
            for k in range(_NO):
                fk = off + k
                row = rbase + lax.shift_right_logical(fk, 7)
                col = fk & 127
                p.append(plsc.load_gather(psb, [row, col]))
            sg = lambda z: 1.0 / (1.0 + jnp.exp(-z))
            aw = aaw[sl]
            ah = aah[sl]
            px = sg(p[0]) * 2.0 - 0.5
            py = sg(p[1]) * 2.0 - 0.5
            sw = sg(p[2]) * 2.0
            sh = sg(p[3]) * 2.0
            pw = sw * sw * aw
            ph = sh * sh * ah
            tbx = atbx[sl]
            tby = atby[sl]
            tw = atw[sl]
            th = ath[sl]
            eps = jnp.float32(1e-7)
            b1x1, b1x2 = px - pw * 0.5, px + pw * 0.5
            b1y1, b1y2 = py - ph * 0.5, py + ph * 0.5
            b2x1, b2x2 = tbx - tw * 0.5, tbx + tw * 0.5
            b2y1, b2y2 = tby - th * 0.5, tby + th * 0.5
            iw = jnp.maximum(jnp.minimum(b1x2, b2x2) - jnp.maximum(b1x1, b2x1), 0.0)
            ih = jnp.maximum(jnp.minimum(b1y2, b2y2) - jnp.maximum(b1y1, b2y1), 0.0)
            inter = iw * ih
            w1_, h1_ = b1x2 - b1x1, b1y2 - b1y1 + eps
            w2_, h2_ = b2x2 - b2x1, b2y2 - b2y1 + eps
            union = w1_ * h1_ + w2_ * h2_ - inter + eps
            iou = inter / union
            vf = aval[sl]
            lbox_acc = lbox_acc + (1.0 - iou) * vf
            cnt_acc = cnt_acc + vf
            v = jnp.maximum(iou, 0.0)
            x4vb[sl] = p[4] * v * vf
            for k in range(7):
                clsb[k, sl] = p[5 + k]
            return lbox_acc, cnt_acc

        carry = (zf, zf)
        for b in range(_NBATCH):
            if b + 1 < _NBATCH:
                nxt = _fire(b + 1)
            else:
                nxt = None
            for cp in pending:
                cp.wait()
            pending = nxt

            @pl.loop(0, _BCH, init_carry=carry)
            def _batch(i, c):
                return _pass2((b * _BCH + i, c))

            carry = _batch
        lbox_acc, cnt_acc = carry

        pltpu.sync_copy(acell, cell_o.at[pl.ds(s * _M + base, _CPT)])
        pltpu.sync_copy(x4vb, x4v_o.at[pl.ds(s * _M + base, _CPT)])
        for k in range(8):
            pltpu.sync_copy(clsb.at[k],
                            cls_o.at[pl.ds((s * 8 + k) * _M + base, _CPT)])
        stage[pl.ds(0, 16)] = lbox_acc
        stage[pl.ds(16, 16)] = cnt_acc
        for z in range(2, 8):
            stage[pl.ds(z * 16, 16)] = zf
        pltpu.sync_copy(stage, part_o.at[pl.ds((s * _NW + wid) * 128, 128)])


def _phase_b_body(cell_hbm, x4v_hbm, corr_o, L, cbuf, vbuf, stg):
    wid = lax.axis_index("s") * 2 + lax.axis_index("c")
    for z in range(8):
        stg[pl.ds(z * 16, 16)] = jnp.zeros((16,), _F32)
    pltpu.sync_copy(stg, corr_o.at[pl.ds(wid * 128, 128)])

    @pl.when(wid < 3)
    def _work():
        @pl.loop(0, _CELLS // 16)
        def _zero(i):
            L[pl.ds(i * 16, 16)] = jnp.zeros((16,), _F32)

        nch = 8
        csz = _M // nch
        for ch in range(nch):
            pltpu.sync_copy(cell_hbm.at[pl.ds(wid * _M + ch * csz, csz)], cbuf)
            pltpu.sync_copy(x4v_hbm.at[pl.ds(wid * _M + ch * csz, csz)], vbuf)

            @pl.loop(0, csz // 16)
            def _scatter(i):
                sl = pl.ds(i * 16, 16)
                c = cbuf[sl]
                vv = vbuf[sl]
                plsc.store_scatter(L, [c], vv, mask=c >= 0)

        @pl.loop(0, _CELLS // 16, init_carry=jnp.zeros((16,), _F32))
        def _reduce(i, acc):
            return acc + L[pl.ds(i * 16, 16)]

        stg[pl.ds(0, 16)] = _reduce
        pltpu.sync_copy(stg, corr_o.at[pl.ds(wid * 128, 128)])


def _softsum_channel4(pview, rows_blk):
    rows, width = pview.shape
    steps = rows // rows_blk

    def body(x_ref, o_ref):
        i = pl.program_id(0)

        @pl.when(i == 0)
        def _init():
            o_ref[...] = jnp.zeros((1, 1), _F32)

        x = x_ref[...]
        r = lax.broadcasted_iota(_I32, x.shape, 0)
        c = lax.broadcasted_iota(_I32, x.shape, 1)
        e = (i * rows_blk + r) * 128 + c
        ch = e - lax.div(e, _NO) * _NO
        sp = jnp.maximum(x, 0.0) + jnp.log1p(jnp.exp(-jnp.abs(x)))
        o_ref[...] += jnp.sum(jnp.where(ch == 4, sp, 0.0)).reshape(1, 1)

    out = pl.pallas_call(
        body,
        grid=(steps,),
        in_specs=[pl.BlockSpec((rows_blk, width), lambda i: (i, 0))],
        out_specs=pl.BlockSpec((1, 1), lambda i: (0, 0)),
        out_shape=jax.ShapeDtypeStruct((1, 1), _F32),
    )(pview)
    return out[0, 0]


def _cls_sums(cls_rows):
    def body(x_ref, o_ref):
        x = x_ref[0]
        l = x[:7, :]
        extra = x[7, :]
        vf = jnp.where(extra >= 0.0, 1.0, 0.0).astype(_F32)
        c = jnp.maximum(extra, 0.0).astype(_I32)
        oh = (lax.broadcasted_iota(_I32, (7, _M), 0) == c[None, :]).astype(_F32)
        ce = jnp.maximum(l, 0.0) - l * oh + jnp.log1p(jnp.exp(-jnp.abs(l)))
        ssum = jnp.sum(ce * vf[None, :])
        i = pl.program_id(0)

        @pl.when(i == 0)
        def _init():
            o_ref[...] = jnp.zeros((8, 128), _F32)

        rowi = lax.broadcasted_iota(_I32, (8, 128), 0)
        lanei = lax.broadcasted_iota(_I32, (8, 128), 1)
        o_ref[...] += jnp.where((rowi == i) & (lanei == 0), ssum, 0.0)

    out = pl.pallas_call(
        body,
        grid=(3,),
        in_specs=[pl.BlockSpec((1, 8, _M), lambda i: (i, 0, 0))],
        out_specs=pl.BlockSpec((8, 128), lambda i: (0, 0)),
        out_shape=jax.ShapeDtypeStruct((8, 128), _F32),
    )(cls_rows)
    return out[:3, 0]


def kernel(p3, p4, p5, targets, img_size):
    preds = (p3, p4, p5)
    pviews = tuple(p.reshape(-1, 128) for p in preds)
    tflat = targets.reshape(-1).astype(_F32)
    tflat = jnp.concatenate([tflat, jnp.zeros((12032 - 6 * _NT,), _F32)])

    mesh = plsc.VectorSubcoreMesh(core_axis_name="c", subcore_axis_name="s")
    phase_a = pl.kernel(
        _phase_a_body,
        out_type=[
            jax.ShapeDtypeStruct((3 * _M,), _I32),
            jax.ShapeDtypeStruct((3 * _M,), _F32),
            jax.ShapeDtypeStruct((3 * 8 * _M,), _F32),
            jax.ShapeDtypeStruct((3 * _NW * 128,), _F32),
        ],
        mesh=mesh,
        compiler_params=pltpu.CompilerParams(needs_layout_passes=False),
        scratch_types=[
            pltpu.VMEM((12032,), _F32),
            pltpu.VMEM((_CHUNKS * 32,), _I32),
            pltpu.VMEM((_CPT,), _I32),
            pltpu.VMEM((2 * _BROWS, 128), _F32),
            pltpu.VMEM((_CPT,), _F32),
            pltpu.VMEM((_CPT,), _F32),
            pltpu.VMEM((_CPT,), _F32),
            pltpu.VMEM((_CPT,), _F32),
            pltpu.VMEM((_CPT,), _F32),
            pltpu.VMEM((_CPT,), _F32),
            pltpu.VMEM((_CPT,), _F32),
            pltpu.VMEM((_CPT,), _I32),
            pltpu.VMEM((_CPT,), _F32),
            pltpu.VMEM((8, _CPT), _F32),
            pltpu.VMEM((128,), _F32),
            pltpu.SemaphoreType.DMA,
        ],
    )
    cells, x4v, cls_flat, part_flat = phase_a(tflat, *pviews)
    cls_rows = cls_flat.reshape(3, 8, _M)
    part = part_flat.reshape(3, _NW, 128)

    phase_b = pl.kernel(
        _phase_b_body,
        out_type=[jax.ShapeDtypeStruct((_NW * 128,), _F32)],
        mesh=mesh,
        compiler_params=pltpu.CompilerParams(needs_layout_passes=False),
        scratch_types=[
            pltpu.VMEM((_CELLS,), _F32),
            pltpu.VMEM((_M // 8,), _I32),
            pltpu.VMEM((_M // 8,), _F32),
            pltpu.VMEM((128,), _F32),
        ],
    )
    (corr,) = phase_b(cells, x4v)
    corr = corr.reshape(_NW, 128)

    softs = [
        _softsum_channel4(pviews[0], 3600),
        _softsum_channel4(pviews[1], 3600),
        _softsum_channel4(pviews[2], 3600),
    ]
    clsums = _cls_sums(cls_rows)

    lbox_s = part[:, :, 0:16].sum(axis=(1, 2))
    cnt_s = part[:, :, 16:32].sum(axis=(1, 2))
    corr_s = corr[:3, :16].sum(axis=1)
    denom = jnp.maximum(cnt_s, 1.0)
    pos = cnt_s > 0
    lbox = jnp.where(pos, lbox_s / denom, 0.0).sum() * _BOX_W
    lcls = jnp.where(pos, clsums / (denom * 7.0), 0.0).sum() * _CLS_W
    lobj = sum(
        (softs[s] - corr_s[s]) / _NCELLS[s] * _BALANCE[s] for s in range(3)
    ) * _OBJ_W
    total = lbox + lobj + lcls
    total = total + jnp.asarray(img_size, dtype=total.dtype) * 0.0
    return (total, jnp.asarray(lbox, _F32), jnp.asarray(lobj, _F32),
            jnp.asarray(lcls, _F32))
